# baseline (speedup 1.0000x reference)
import jax
import jax.numpy as jnp
from jax.experimental import pallas as pl
from jax.experimental.pallas import tpu as pltpu

_IMG_H = 1024.0
_IMG_W = 1024.0

_SIN_COEFFS = (33.1688117980957, -74.67622375488281, 81.40013885498047,
               -41.333251953125, 6.283088684082031)
_COS_COEFFS = (46.310630798339844, -82.70143127441406, 64.71440124511719,
               -19.732797622680664, 0.9999710917472839)


def _sincos(t):
    r = t - jnp.round(t)
    u = r * r
    s = _SIN_COEFFS[0]
    for cc in _SIN_COEFFS[1:]:
        s = s * u + cc
    s = s * r
    c = _COS_COEFFS[0]
    for cc in _COS_COEFFS[1:]:
        c = c * u + cc
    return s, c


def _body(pad_ref, ct_ref, l_ref, g_ref, pe0_ref, pe1_ref, pe2_ref, o_ref):
    d = g_ref.shape[1]
    bb, np_ = l_ref.shape
    f32 = jnp.float32
    sc = 2.0 / _IMG_W
    off = 0.5 * sc - 1.0
    cl = (ct_ref[...] * sc + off).reshape(2, bb * np_)
    cb = cl.astype(jnp.bfloat16)
    gb = g_ref[...].astype(jnp.bfloat16)
    t2 = jax.lax.dot_general(cb, gb, (((0,), (0,)), ((), ())),
                             preferred_element_type=f32)
    t = t2.reshape(bb, np_, d)
    s, c = _sincos(t)
    lab = l_ref[...][:, :, None]
    m0 = lab == 0
    m1 = lab == 1
    m2 = lab == 2
    pe_lo = [ref[0, :d][None, None, :] for ref in (pe0_ref, pe1_ref, pe2_ref)]
    pe_hi = [ref[0, d:][None, None, :] for ref in (pe0_ref, pe1_ref, pe2_ref)]
    sel_lo = jnp.where(m0, pe_lo[0], jnp.where(m1, pe_lo[1],
                       jnp.where(m2, pe_lo[2], 0.0)))
    sel_hi = jnp.where(m0, pe_hi[0], jnp.where(m1, pe_hi[1],
                       jnp.where(m2, pe_hi[2], 0.0)))
    o_ref[:, :np_, :d] = s + sel_lo
    o_ref[:, :np_, d:] = c + sel_hi
    g0 = g_ref[0, :].astype(jnp.bfloat16).astype(f32)
    g1 = g_ref[1, :].astype(jnp.bfloat16).astype(f32)
    t_pad = (-1.0) * g0 + (-1.0) * g1
    s_pad, c_pad = _sincos(t_pad)
    plab = pad_ref[0, 0]
    row_lo = s_pad + jnp.where(plab == 0, pe_lo[0][0, 0], jnp.where(
        plab == 1, pe_lo[1][0, 0], jnp.where(plab == 2, pe_lo[2][0, 0], 0.0)))
    row_hi = c_pad + jnp.where(plab == 0, pe_hi[0][0, 0], jnp.where(
        plab == 1, pe_hi[1][0, 0], jnp.where(plab == 2, pe_hi[2][0, 0], 0.0)))
    o_ref[:, np_:, :d] = jnp.broadcast_to(row_lo[None, None, :], (bb, 1, d))
    o_ref[:, np_:, d:] = jnp.broadcast_to(row_hi[None, None, :], (bb, 1, d))


def kernel(points, labels, pad, gauss, pe0, pe1, pe2):
    B, P, _ = points.shape
    D = gauss.shape[1]
    P1 = P + 1
    neg_pad = (-jnp.asarray(pad, jnp.int32)).reshape(1, 1)
    coords_t = jnp.transpose(points, (2, 0, 1))

    BB = 128
    out = pl.pallas_call(
        _body,
        grid=(B // BB,),
        in_specs=[
            pl.BlockSpec(memory_space=pltpu.SMEM),
            pl.BlockSpec((2, BB, P), lambda i: (0, i, 0)),
            pl.BlockSpec((BB, P), lambda i: (i, 0)),
            pl.BlockSpec((2, D), lambda i: (0, 0)),
            pl.BlockSpec((1, 2 * D), lambda i: (0, 0)),
            pl.BlockSpec((1, 2 * D), lambda i: (0, 0)),
            pl.BlockSpec((1, 2 * D), lambda i: (0, 0)),
        ],
        out_specs=pl.BlockSpec((BB, P1, 2 * D), lambda i: (i, 0, 0)),
        out_shape=jax.ShapeDtypeStruct((B, P1, 2 * D), points.dtype),
        compiler_params=pltpu.CompilerParams(dimension_semantics=("parallel",)),
    )(neg_pad, coords_t, labels, gauss, pe0, pe1, pe2)
    return out


# speedup vs baseline: 1.0026x; 1.0026x over previous
import jax
import jax.numpy as jnp
from jax.experimental import pallas as pl
from jax.experimental.pallas import tpu as pltpu

_IMG_H = 1024.0
_IMG_W = 1024.0

_SIN_COEFFS = (33.1688117980957, -74.67622375488281, 81.40013885498047,
               -41.333251953125, 6.283088684082031)
_COS_COEFFS = (46.310630798339844, -82.70143127441406, 64.71440124511719,
               -19.732797622680664, 0.9999710917472839)


def _sincos(t):
    r = t - jnp.round(t)
    u = r * r
    s = _SIN_COEFFS[0]
    for cc in _SIN_COEFFS[1:]:
        s = s * u + cc
    s = s * r
    c = _COS_COEFFS[0]
    for cc in _COS_COEFFS[1:]:
        c = c * u + cc
    return s, c


def _body(pad_ref, ct_ref, l_ref, g_ref, pe0_ref, pe1_ref, pe2_ref, o_ref):
    d = g_ref.shape[1]
    bb, np_ = l_ref.shape
    f32 = jnp.float32
    sc = 2.0 / _IMG_W
    off = 0.5 * sc - 1.0
    cl = ct_ref[...] * sc + off
    cb = cl.astype(jnp.bfloat16)
    gb = g_ref[...].astype(jnp.bfloat16)
    t2 = jax.lax.dot_general(cb, gb, (((0,), (0,)), ((), ())),
                             preferred_element_type=f32)
    t = t2.reshape(bb, np_, d)
    s, c = _sincos(t)
    lab = l_ref[...][:, :, None]
    m0 = lab == 0
    m1 = lab == 1
    m2 = lab == 2
    pe_lo = [ref[0, :d][None, None, :] for ref in (pe0_ref, pe1_ref, pe2_ref)]
    pe_hi = [ref[0, d:][None, None, :] for ref in (pe0_ref, pe1_ref, pe2_ref)]
    sel_lo = jnp.where(m0, pe_lo[0], jnp.where(m1, pe_lo[1],
                       jnp.where(m2, pe_lo[2], 0.0)))
    sel_hi = jnp.where(m0, pe_hi[0], jnp.where(m1, pe_hi[1],
                       jnp.where(m2, pe_hi[2], 0.0)))
    o_ref[:, :np_, :d] = s + sel_lo
    o_ref[:, :np_, d:] = c + sel_hi
    g0 = g_ref[0, :].astype(jnp.bfloat16).astype(f32)
    g1 = g_ref[1, :].astype(jnp.bfloat16).astype(f32)
    t_pad = (-1.0) * g0 + (-1.0) * g1
    s_pad, c_pad = _sincos(t_pad)
    plab = pad_ref[0, 0]
    row_lo = s_pad + jnp.where(plab == 0, pe_lo[0][0, 0], jnp.where(
        plab == 1, pe_lo[1][0, 0], jnp.where(plab == 2, pe_lo[2][0, 0], 0.0)))
    row_hi = c_pad + jnp.where(plab == 0, pe_hi[0][0, 0], jnp.where(
        plab == 1, pe_hi[1][0, 0], jnp.where(plab == 2, pe_hi[2][0, 0], 0.0)))
    o_ref[:, np_:, :d] = jnp.broadcast_to(row_lo[None, None, :], (bb, 1, d))
    o_ref[:, np_:, d:] = jnp.broadcast_to(row_hi[None, None, :], (bb, 1, d))


def kernel(points, labels, pad, gauss, pe0, pe1, pe2):
    B, P, _ = points.shape
    D = gauss.shape[1]
    P1 = P + 1
    neg_pad = (-jnp.asarray(pad, jnp.int32)).reshape(1, 1)
    coords_t = jnp.transpose(points, (2, 0, 1)).reshape(2, B * P)

    BB = 128
    out = pl.pallas_call(
        _body,
        grid=(B // BB,),
        in_specs=[
            pl.BlockSpec(memory_space=pltpu.SMEM),
            pl.BlockSpec((2, BB * P), lambda i: (0, i)),
            pl.BlockSpec((BB, P), lambda i: (i, 0)),
            pl.BlockSpec((2, D), lambda i: (0, 0)),
            pl.BlockSpec((1, 2 * D), lambda i: (0, 0)),
            pl.BlockSpec((1, 2 * D), lambda i: (0, 0)),
            pl.BlockSpec((1, 2 * D), lambda i: (0, 0)),
        ],
        out_specs=pl.BlockSpec((BB, P1, 2 * D), lambda i: (i, 0, 0)),
        out_shape=jax.ShapeDtypeStruct((B, P1, 2 * D), points.dtype),
        compiler_params=pltpu.CompilerParams(dimension_semantics=("parallel",)),
    )(neg_pad, coords_t, labels, gauss, pe0, pe1, pe2)
    return out
